# baseline (speedup 1.0000x reference)
import functools

import jax
import jax.numpy as jnp
from jax import lax
from jax.experimental import pallas as pl
from jax.experimental.pallas import tpu as pltpu
from jax.experimental.pallas import tpu_sc as plsc

B, L = 4096, 50
DT, DC = 64, 32
NCAT = 8
NUMF, NUMH = 16, 64
HID, NCLS = 512, 100
FUSION = 2 * DT + NCAT * DC + NUMH

NC, NS = 2, 16
NW = NC * NS
BPW = B // NW
NBUF = 4
GROUPS = BPW // NBUF
LPAD = 64
VT, VC = 100000, 100000
VHALF = VT // 2
VQUART = VC // 4

_mesh = plsc.VectorSubcoreMesh(
    core_axis_name="c", subcore_axis_name="s", num_cores=NC, num_subcores=NS)
_sc_params = pltpu.CompilerParams()


@functools.partial(
    pl.kernel,
    out_type=jax.ShapeDtypeStruct((B, 2 * DT), jnp.float32),
    mesh=_mesh,
    compiler_params=_sc_params,
    scratch_types=[
        pltpu.VMEM((BPW * LPAD,), jnp.int32),
        pltpu.VMEM((BPW * LPAD,), jnp.int32),
        pltpu.VMEM((BPW * LPAD,), jnp.int32),
        pltpu.VMEM((L, 128), jnp.float32),
        pltpu.VMEM((L, 128), jnp.float32),
        pltpu.VMEM((L, 128), jnp.float32),
        pltpu.VMEM((L, 128), jnp.float32),
        pltpu.VMEM((BPW, 2 * DT), jnp.float32),
        pltpu.SemaphoreType.DMA,
        pltpu.SemaphoreType.DMA,
        pltpu.SemaphoreType.DMA,
        pltpu.SemaphoreType.DMA,
    ],
)
def _sc_text(tt_hbm, tq_hbm, embt_hbm, embq_hbm, text_out,
             idx_v, q_v, cb_v, r0, r1, r2, r3, stage_t, s0, s1, s2, s3):
    wid = lax.axis_index("s") * NC + lax.axis_index("c")
    base = wid * BPW
    rows = (r0, r1, r2, r3)
    sems = (s0, s1, s2, s3)

    def pool_table(src_hbm, tbl_hbm, col0):
        pltpu.sync_copy(src_hbm.at[pl.ds(base * LPAD, BPW * LPAD)], idx_v)

        def transform(i, carry):
            r = idx_v[pl.ds(i * 16, 16)]
            hi = r >= VHALF
            q_v[pl.ds(i * 16, 16)] = r - jnp.where(hi, VHALF, 0)
            cb_v[pl.ds(i * 16, 16)] = jnp.where(hi, DT, 0)
            return carry

        lax.fori_loop(0, BPW * LPAD // 16, transform, 0)

        for b in range(NBUF):
            pltpu.make_async_copy(
                tbl_hbm.at[q_v.at[pl.ds(b * LPAD, L)]], rows[b],
                sems[b]).start()

        def group(i, carry):
            for b in range(NBUF):
                s = i * NBUF + b
                pltpu.make_async_copy(
                    tbl_hbm.at[q_v.at[pl.ds(s * LPAD, L)]], rows[b],
                    sems[b]).wait()
                acc = None
                for g in range((L + 15) // 16):
                    cbvec = cb_v[pl.ds(s * LPAD + g * 16, 16)]
                    for jj in range(min(16, L - g * 16)):
                        j = g * 16 + jj
                        cbj = cbvec[jj]
                        cur = [rows[b][j, pl.ds(cbj + k * 16, 16)]
                               for k in range(DT // 16)]
                        if acc is None:
                            acc = cur
                        else:
                            acc = [a + c for a, c in zip(acc, cur)]
                for k in range(DT // 16):
                    stage_t[s, pl.ds(col0 + k * 16, 16)] = acc[k] * (1.0 / L)
                nxt = s + NBUF

                @pl.when(nxt < BPW)
                def _():
                    pltpu.make_async_copy(
                        tbl_hbm.at[q_v.at[pl.ds(nxt * LPAD, L)]], rows[b],
                        sems[b]).start()
            return carry

        lax.fori_loop(0, GROUPS, group, 0)

    pool_table(tt_hbm, embt_hbm, 0)
    pool_table(tq_hbm, embq_hbm, DT)
    pltpu.sync_copy(stage_t, text_out.at[pl.ds(base, BPW), :])


@functools.partial(
    pl.kernel,
    out_type=jax.ShapeDtypeStruct((2, B, 128), jnp.float32),
    mesh=_mesh,
    compiler_params=_sc_params,
    scratch_types=[
        pltpu.VMEM((NCAT, BPW), jnp.int32),
        pltpu.VMEM((NCAT, BPW), jnp.int32),
        pltpu.VMEM((BPW, 128), jnp.float32),
        pltpu.VMEM((BPW, 128), jnp.float32),
        pltpu.VMEM((BPW, 128), jnp.float32),
        pltpu.VMEM((BPW, 128), jnp.float32),
        pltpu.VMEM((BPW, 4 * DC), jnp.float32),
        pltpu.VMEM((BPW, 4 * DC), jnp.float32),
        pltpu.SemaphoreType.DMA,
        pltpu.SemaphoreType.DMA,
    ],
)
def _sc_cats(cats_hbm, ec0, ec1, ec2, ec3, ec4, ec5, ec6, ec7, cat_out,
             cidx_v, ccb_v, c0, c1, c2, c3, stage_c0, stage_c1, cs0, cs1):
    wid = lax.axis_index("s") * NC + lax.axis_index("c")
    base = wid * BPW
    ctbls = (ec0, ec1, ec2, ec3, ec4, ec5, ec6, ec7)
    crows = (c0, c1, c2, c3)
    stages = (stage_c0, stage_c1)

    for t in range(NCAT):
        pltpu.make_async_copy(
            cats_hbm.at[pl.ds(t * B + base, BPW)], cidx_v.at[t], cs1).start()
    for t in range(NCAT):
        pltpu.make_async_copy(
            cats_hbm.at[pl.ds(t * B + base, BPW)], cidx_v.at[t], cs1).wait()

    def transform(i, carry):
        r = cidx_v[i // (BPW // 16), pl.ds((i % (BPW // 16)) * 16, 16)]
        sub = (jnp.where(r >= VQUART, VQUART, 0)
               + jnp.where(r >= 2 * VQUART, VQUART, 0)
               + jnp.where(r >= 3 * VQUART, VQUART, 0))
        cidx_v[i // (BPW // 16), pl.ds((i % (BPW // 16)) * 16, 16)] = r - sub
        ccb_v[i // (BPW // 16), pl.ds((i % (BPW // 16)) * 16, 16)] = (
            jnp.where(r >= VQUART, DC, 0)
            + jnp.where(r >= 2 * VQUART, DC, 0)
            + jnp.where(r >= 3 * VQUART, DC, 0))
        return carry

    lax.fori_loop(0, NCAT * (BPW // 16), transform, 0)

    for w in range(2):
        for t in range(4):
            tt = w * 4 + t
            pltpu.make_async_copy(
                ctbls[tt].at[cidx_v.at[tt]], crows[t], cs0).start()
        for t in range(4):
            tt = w * 4 + t
            pltpu.make_async_copy(
                ctbls[tt].at[cidx_v.at[tt]], crows[t], cs0).wait()

        def cat_assemble(g, carry):
            for t in range(4):
                tt = w * 4 + t
                cbvec = ccb_v[tt, pl.ds(g * 16, 16)]
                for ss in range(16):
                    s = g * 16 + ss
                    cb = cbvec[ss]
                    for k in range(DC // 16):
                        stages[w][s, pl.ds(t * DC + k * 16, 16)] = (
                            crows[t][s, pl.ds(cb + k * 16, 16)])
            return carry

        lax.fori_loop(0, BPW // 16, cat_assemble, 0)
        pltpu.sync_copy(stages[w], cat_out.at[w, pl.ds(base, BPW), :])


def _compact(table, d):
    v = table.shape[0]
    rp = 128 // d
    vq = v // rp
    br = 2000 if rp == 2 else 1000

    def body(*refs):
        o_ref = refs[-1]
        o_ref[:] = jnp.concatenate([r[:] for r in refs[:-1]], axis=1)

    nblk = vq // br
    return pl.pallas_call(
        body,
        grid=(nblk,),
        in_specs=[pl.BlockSpec((br, d),
                               functools.partial(
                                   lambda c, i: (i + c * nblk, 0), c))
                  for c in range(rp)],
        out_specs=pl.BlockSpec((br, 128), lambda i: (i, 0)),
        out_shape=jax.ShapeDtypeStruct((vq, 128), jnp.float32),
    )(*([table] * rp))


def _tc_mlp(feat, cat, num, wn, bn, w1, b1, w2, b2, out):
    nm = jnp.maximum(
        jnp.dot(num[:], wn[:], preferred_element_type=jnp.float32) + bn[:],
        0.0)
    f = jnp.concatenate([feat[:], cat[0], cat[1], nm], axis=1)
    h = jnp.maximum(
        jnp.dot(f, w1[:], preferred_element_type=jnp.float32) + b1[:], 0.0)
    out[:] = jnp.dot(h, w2[:], preferred_element_type=jnp.float32) + b2[:]


BM = 512


def _mlp_call(feat, cat, num, wn, bn, w1, b1, w2, b2):
    return pl.pallas_call(
        _tc_mlp,
        grid=(B // BM,),
        in_specs=[
            pl.BlockSpec((BM, 2 * DT), lambda i: (i, 0)),
            pl.BlockSpec((2, BM, 128), lambda i: (0, i, 0)),
            pl.BlockSpec((BM, NUMF), lambda i: (i, 0)),
            pl.BlockSpec((NUMF, NUMH), lambda i: (0, 0)),
            pl.BlockSpec((1, NUMH), lambda i: (0, 0)),
            pl.BlockSpec((FUSION, HID), lambda i: (0, 0)),
            pl.BlockSpec((1, HID), lambda i: (0, 0)),
            pl.BlockSpec((HID, NCLS), lambda i: (0, 0)),
            pl.BlockSpec((1, NCLS), lambda i: (0, 0)),
        ],
        out_specs=pl.BlockSpec((BM, NCLS), lambda i: (i, 0)),
        out_shape=jax.ShapeDtypeStruct((B, NCLS), jnp.float32),
    )(feat, cat, num, wn, bn, w1, b1, w2, b2)


def kernel(text_title, text_query, cat0, cat1, cat2, cat3, cat4, cat5, cat6,
           cat7, numerical_inputs, emb_title, emb_query, emb_cat0, emb_cat1,
           emb_cat2, emb_cat3, emb_cat4, emb_cat5, emb_cat6, emb_cat7,
           W_num, b_num, W1, b1, W2, b2):
    tt = jnp.pad(text_title.astype(jnp.int32),
                 ((0, 0), (0, LPAD - L))).reshape(-1)
    tq = jnp.pad(text_query.astype(jnp.int32),
                 ((0, 0), (0, LPAD - L))).reshape(-1)
    cats = jnp.stack([cat0, cat1, cat2, cat3, cat4, cat5, cat6, cat7]
                     ).astype(jnp.int32).reshape(-1)
    embt = _compact(emb_title, DT)
    embq = _compact(emb_query, DT)
    ecs = [_compact(e, DC) for e in
           (emb_cat0, emb_cat1, emb_cat2, emb_cat3,
            emb_cat4, emb_cat5, emb_cat6, emb_cat7)]
    feat = _sc_text(tt, tq, embt, embq)
    cat = _sc_cats(cats, *ecs)
    return _mlp_call(feat, cat, numerical_inputs,
                     W_num, b_num.reshape(1, NUMH),
                     W1, b1.reshape(1, HID),
                     W2, b2.reshape(1, NCLS))


# speedup vs baseline: 1.4128x; 1.4128x over previous
import functools

import jax
import jax.numpy as jnp
from jax import lax
from jax.experimental import pallas as pl
from jax.experimental.pallas import tpu as pltpu
from jax.experimental.pallas import tpu_sc as plsc
from jax.experimental.layout import Format, Layout, with_layout_constraint

B, L = 4096, 50
DT, DC = 64, 32
NCAT = 8
NUMF, NUMH = 16, 64
HID, NCLS = 512, 100
FUSION = 2 * DT + NCAT * DC + NUMH

NC, NS = 2, 16
NW = NC * NS
BPW = B // NW
NBUF = 4
GROUPS = BPW // NBUF
LPAD = 56

_mesh = plsc.VectorSubcoreMesh(
    core_axis_name="c", subcore_axis_name="s", num_cores=NC, num_subcores=NS)
_sc_params = pltpu.CompilerParams(use_tc_tiling_on_sc=False)
_ROWMAJOR = Layout(major_to_minor=(1, 0), tiling=((8, 128),))


@functools.partial(
    pl.kernel,
    out_type=jax.ShapeDtypeStruct((B, 2 * DT), jnp.float32),
    mesh=_mesh,
    compiler_params=_sc_params,
    scratch_types=[
        pltpu.VMEM((BPW * LPAD,), jnp.int32),
        pltpu.VMEM((L, DT), jnp.float32),
        pltpu.VMEM((L, DT), jnp.float32),
        pltpu.VMEM((L, DT), jnp.float32),
        pltpu.VMEM((L, DT), jnp.float32),
        pltpu.VMEM((BPW, 2 * DT), jnp.float32),
        pltpu.SemaphoreType.DMA,
        pltpu.SemaphoreType.DMA,
        pltpu.SemaphoreType.DMA,
        pltpu.SemaphoreType.DMA,
    ],
)
def _sc_text(tt_hbm, tq_hbm, embt_hbm, embq_hbm, text_out,
             idx_v, r0, r1, r2, r3, stage_t, s0, s1, s2, s3):
    wid = lax.axis_index("s") * NC + lax.axis_index("c")
    base = wid * BPW
    rows = (r0, r1, r2, r3)
    sems = (s0, s1, s2, s3)

    def pool_table(src_hbm, tbl_hbm, col0):
        pltpu.sync_copy(src_hbm.at[pl.ds(base * LPAD, BPW * LPAD)], idx_v)
        for b in range(NBUF):
            pltpu.make_async_copy(
                tbl_hbm.at[idx_v.at[pl.ds(b * LPAD, L)]], rows[b],
                sems[b]).start()

        def group(i, carry):
            for b in range(NBUF):
                s = i * NBUF + b
                pltpu.make_async_copy(
                    tbl_hbm.at[idx_v.at[pl.ds(s * LPAD, L)]], rows[b],
                    sems[b]).wait()
                acc = [rows[b][0, pl.ds(k * 16, 16)] for k in range(DT // 16)]
                for j in range(1, L):
                    for k in range(DT // 16):
                        acc[k] = acc[k] + rows[b][j, pl.ds(k * 16, 16)]
                for k in range(DT // 16):
                    stage_t[s, pl.ds(col0 + k * 16, 16)] = acc[k] * (1.0 / L)
                nxt = s + NBUF

                @pl.when(nxt < BPW)
                def _():
                    pltpu.make_async_copy(
                        tbl_hbm.at[idx_v.at[pl.ds(nxt * LPAD, L)]], rows[b],
                        sems[b]).start()
            return carry

        lax.fori_loop(0, GROUPS, group, 0)

    pool_table(tt_hbm, embt_hbm, 0)
    pool_table(tq_hbm, embq_hbm, DT)
    pltpu.sync_copy(stage_t, text_out.at[pl.ds(base, BPW), :])


@functools.partial(
    pl.kernel,
    out_type=jax.ShapeDtypeStruct((2, B, 128), jnp.float32),
    mesh=_mesh,
    compiler_params=_sc_params,
    scratch_types=[
        pltpu.VMEM((NCAT, BPW), jnp.int32),
        pltpu.VMEM((BPW, DC), jnp.float32),
        pltpu.VMEM((BPW, DC), jnp.float32),
        pltpu.VMEM((BPW, DC), jnp.float32),
        pltpu.VMEM((BPW, DC), jnp.float32),
        pltpu.VMEM((BPW, DC), jnp.float32),
        pltpu.VMEM((BPW, DC), jnp.float32),
        pltpu.VMEM((BPW, DC), jnp.float32),
        pltpu.VMEM((BPW, DC), jnp.float32),
        pltpu.VMEM((BPW, 4 * DC), jnp.float32),
        pltpu.VMEM((BPW, 4 * DC), jnp.float32),
        pltpu.SemaphoreType.DMA,
        pltpu.SemaphoreType.DMA,
    ],
)
def _sc_cats(cats_hbm, ec0, ec1, ec2, ec3, ec4, ec5, ec6, ec7, cat_out,
             cidx_v, c0, c1, c2, c3, c4, c5, c6, c7,
             stage_c0, stage_c1, cs0, cs1):
    wid = lax.axis_index("s") * NC + lax.axis_index("c")
    base = wid * BPW
    ctbls = (ec0, ec1, ec2, ec3, ec4, ec5, ec6, ec7)
    crows = (c0, c1, c2, c3, c4, c5, c6, c7)
    stages = (stage_c0, stage_c1)

    for t in range(NCAT):
        pltpu.make_async_copy(
            cats_hbm.at[pl.ds(t * B + base, BPW)], cidx_v.at[t], cs1).start()
    for t in range(NCAT):
        pltpu.make_async_copy(
            cats_hbm.at[pl.ds(t * B + base, BPW)], cidx_v.at[t], cs1).wait()
    for t in range(NCAT):
        pltpu.make_async_copy(
            ctbls[t].at[cidx_v.at[t]], crows[t], cs0).start()
    for t in range(NCAT):
        pltpu.make_async_copy(
            ctbls[t].at[cidx_v.at[t]], crows[t], cs0).wait()

    def cat_assemble(s, carry):
        for t in range(NCAT):
            for k in range(DC // 16):
                stages[t // 4][s, pl.ds((t % 4) * DC + k * 16, 16)] = (
                    crows[t][s, pl.ds(k * 16, 16)])
        return carry

    lax.fori_loop(0, BPW, cat_assemble, 0)
    pltpu.sync_copy(stage_c0, cat_out.at[0, pl.ds(base, BPW), :])
    pltpu.sync_copy(stage_c1, cat_out.at[1, pl.ds(base, BPW), :])


def _tc_mlp(feat, cat, num, wn, bn, w1, b1, w2, b2, out):
    nm = jnp.maximum(
        jnp.dot(num[:], wn[:], preferred_element_type=jnp.float32) + bn[:],
        0.0)
    f = jnp.concatenate([feat[:], cat[0], cat[1], nm], axis=1)
    h = jnp.maximum(
        jnp.dot(f, w1[:], preferred_element_type=jnp.float32) + b1[:], 0.0)
    out[:] = jnp.dot(h, w2[:], preferred_element_type=jnp.float32) + b2[:]


BM = 512


def _mlp_call(feat, cat, num, wn, bn, w1, b1, w2, b2):
    return pl.pallas_call(
        _tc_mlp,
        grid=(B // BM,),
        in_specs=[
            pl.BlockSpec((BM, 2 * DT), lambda i: (i, 0)),
            pl.BlockSpec((2, BM, 128), lambda i: (0, i, 0)),
            pl.BlockSpec((BM, NUMF), lambda i: (i, 0)),
            pl.BlockSpec((NUMF, NUMH), lambda i: (0, 0)),
            pl.BlockSpec((1, NUMH), lambda i: (0, 0)),
            pl.BlockSpec((FUSION, HID), lambda i: (0, 0)),
            pl.BlockSpec((1, HID), lambda i: (0, 0)),
            pl.BlockSpec((HID, NCLS), lambda i: (0, 0)),
            pl.BlockSpec((1, NCLS), lambda i: (0, 0)),
        ],
        out_specs=pl.BlockSpec((BM, NCLS), lambda i: (i, 0)),
        out_shape=jax.ShapeDtypeStruct((B, NCLS), jnp.float32),
    )(feat, cat, num, wn, bn, w1, b1, w2, b2)


def kernel(text_title, text_query, cat0, cat1, cat2, cat3, cat4, cat5, cat6,
           cat7, numerical_inputs, emb_title, emb_query, emb_cat0, emb_cat1,
           emb_cat2, emb_cat3, emb_cat4, emb_cat5, emb_cat6, emb_cat7,
           W_num, b_num, W1, b1, W2, b2):
    tt = jnp.pad(text_title.astype(jnp.int32),
                 ((0, 0), (0, LPAD - L))).reshape(-1)
    tq = jnp.pad(text_query.astype(jnp.int32),
                 ((0, 0), (0, LPAD - L))).reshape(-1)
    cats = jnp.stack([cat0, cat1, cat2, cat3, cat4, cat5, cat6, cat7]
                     ).astype(jnp.int32).reshape(-1)
    tables = [with_layout_constraint(t, _ROWMAJOR) for t in
              (emb_title, emb_query, emb_cat0, emb_cat1, emb_cat2, emb_cat3,
               emb_cat4, emb_cat5, emb_cat6, emb_cat7)]
    feat = _sc_text(tt, tq, tables[0], tables[1])
    cat = _sc_cats(cats, *tables[2:])
    return _mlp_call(feat, cat, numerical_inputs,
                     W_num, b_num.reshape(1, NUMH),
                     W1, b1.reshape(1, HID),
                     W2, b2.reshape(1, NCLS))


# speedup vs baseline: 1.5747x; 1.1146x over previous
import functools

import jax
import jax.numpy as jnp
from jax import lax
from jax.experimental import pallas as pl
from jax.experimental.pallas import tpu as pltpu
from jax.experimental.pallas import tpu_sc as plsc

B, L = 4096, 50
DT, DC = 64, 32
NCAT = 8
NUMF, NUMH = 16, 64
HID, NCLS = 512, 100
FUSION = 2 * DT + NCAT * DC + NUMH

NC, NS = 2, 16
NW = NC * NS
BPW = B // NW
NBUF = 4
GROUPS = BPW // NBUF
LPAD = 56

_mesh = plsc.VectorSubcoreMesh(
    core_axis_name="c", subcore_axis_name="s", num_cores=NC, num_subcores=NS)
_sc_params = pltpu.CompilerParams(use_tc_tiling_on_sc=False)


@functools.partial(
    pl.kernel,
    out_type=jax.ShapeDtypeStruct((B, DT), jnp.float32),
    mesh=_mesh,
    compiler_params=_sc_params,
    scratch_types=[
        pltpu.VMEM((BPW * LPAD,), jnp.int32),
        pltpu.VMEM((L, DT), jnp.float32),
        pltpu.VMEM((L, DT), jnp.float32),
        pltpu.VMEM((L, DT), jnp.float32),
        pltpu.VMEM((L, DT), jnp.float32),
        pltpu.VMEM((BPW, DT), jnp.float32),
        pltpu.SemaphoreType.DMA,
        pltpu.SemaphoreType.DMA,
        pltpu.SemaphoreType.DMA,
        pltpu.SemaphoreType.DMA,
    ],
)
def _sc_text(tt_hbm, tbl_hbm, text_out,
             idx_v, r0, r1, r2, r3, stage_t, s0, s1, s2, s3):
    wid = lax.axis_index("s") * NC + lax.axis_index("c")
    base = wid * BPW
    rows = (r0, r1, r2, r3)
    sems = (s0, s1, s2, s3)

    pltpu.sync_copy(tt_hbm.at[pl.ds(base * LPAD, BPW * LPAD)], idx_v)
    for b in range(NBUF):
        pltpu.make_async_copy(
            tbl_hbm.at[idx_v.at[pl.ds(b * LPAD, L)]], rows[b],
            sems[b]).start()

    def group(i, carry):
        for b in range(NBUF):
            s = i * NBUF + b
            pltpu.make_async_copy(
                tbl_hbm.at[idx_v.at[pl.ds(s * LPAD, L)]], rows[b],
                sems[b]).wait()
            acc = [rows[b][0, pl.ds(k * 16, 16)] for k in range(DT // 16)]
            for j in range(1, L):
                for k in range(DT // 16):
                    acc[k] = acc[k] + rows[b][j, pl.ds(k * 16, 16)]
            for k in range(DT // 16):
                stage_t[s, pl.ds(k * 16, 16)] = acc[k] * (1.0 / L)
            nxt = s + NBUF

            @pl.when(nxt < BPW)
            def _():
                pltpu.make_async_copy(
                    tbl_hbm.at[idx_v.at[pl.ds(nxt * LPAD, L)]], rows[b],
                    sems[b]).start()
        return carry

    lax.fori_loop(0, GROUPS, group, 0)
    pltpu.sync_copy(stage_t, text_out.at[pl.ds(base, BPW), :])


@functools.partial(
    pl.kernel,
    out_type=jax.ShapeDtypeStruct((2, B, 128), jnp.float32),
    mesh=_mesh,
    compiler_params=_sc_params,
    scratch_types=[
        pltpu.VMEM((NCAT, BPW), jnp.int32),
        pltpu.VMEM((BPW, DC), jnp.float32),
        pltpu.VMEM((BPW, DC), jnp.float32),
        pltpu.VMEM((BPW, DC), jnp.float32),
        pltpu.VMEM((BPW, DC), jnp.float32),
        pltpu.VMEM((BPW, DC), jnp.float32),
        pltpu.VMEM((BPW, DC), jnp.float32),
        pltpu.VMEM((BPW, DC), jnp.float32),
        pltpu.VMEM((BPW, DC), jnp.float32),
        pltpu.VMEM((BPW, 4 * DC), jnp.float32),
        pltpu.VMEM((BPW, 4 * DC), jnp.float32),
        pltpu.SemaphoreType.DMA,
        pltpu.SemaphoreType.DMA,
    ],
)
def _sc_cats(cats_hbm, ec0, ec1, ec2, ec3, ec4, ec5, ec6, ec7, cat_out,
             cidx_v, c0, c1, c2, c3, c4, c5, c6, c7,
             stage_c0, stage_c1, cs0, cs1):
    wid = lax.axis_index("s") * NC + lax.axis_index("c")
    base = wid * BPW
    ctbls = (ec0, ec1, ec2, ec3, ec4, ec5, ec6, ec7)
    crows = (c0, c1, c2, c3, c4, c5, c6, c7)
    stages = (stage_c0, stage_c1)

    for t in range(NCAT):
        pltpu.make_async_copy(
            cats_hbm.at[pl.ds(t * B + base, BPW)], cidx_v.at[t], cs1).start()
    for t in range(NCAT):
        pltpu.make_async_copy(
            cats_hbm.at[pl.ds(t * B + base, BPW)], cidx_v.at[t], cs1).wait()
    for t in range(NCAT):
        pltpu.make_async_copy(
            ctbls[t].at[cidx_v.at[t]], crows[t], cs0).start()
    for t in range(NCAT):
        pltpu.make_async_copy(
            ctbls[t].at[cidx_v.at[t]], crows[t], cs0).wait()

    def cat_assemble(s, carry):
        for t in range(NCAT):
            for k in range(DC // 16):
                stages[t // 4][s, pl.ds((t % 4) * DC + k * 16, 16)] = (
                    crows[t][s, pl.ds(k * 16, 16)])
        return carry

    lax.fori_loop(0, BPW, cat_assemble, 0)
    pltpu.sync_copy(stage_c0, cat_out.at[0, pl.ds(base, BPW), :])
    pltpu.sync_copy(stage_c1, cat_out.at[1, pl.ds(base, BPW), :])


def _tc_mlp(t1, t2, cat, num, wn, bn, w1, b1, w2, b2, out):
    nm = jnp.maximum(
        jnp.dot(num[:], wn[:], preferred_element_type=jnp.float32) + bn[:],
        0.0)
    f = jnp.concatenate([t1[:], t2[:], cat[0], cat[1], nm], axis=1)
    h = jnp.maximum(
        jnp.dot(f, w1[:], preferred_element_type=jnp.float32) + b1[:], 0.0)
    out[:] = jnp.dot(h, w2[:], preferred_element_type=jnp.float32) + b2[:]


BM = 512


def _mlp_call(t1, t2, cat, num, wn, bn, w1, b1, w2, b2):
    return pl.pallas_call(
        _tc_mlp,
        grid=(B // BM,),
        in_specs=[
            pl.BlockSpec((BM, DT), lambda i: (i, 0)),
            pl.BlockSpec((BM, DT), lambda i: (i, 0)),
            pl.BlockSpec((2, BM, 128), lambda i: (0, i, 0)),
            pl.BlockSpec((BM, NUMF), lambda i: (i, 0)),
            pl.BlockSpec((NUMF, NUMH), lambda i: (0, 0)),
            pl.BlockSpec((1, NUMH), lambda i: (0, 0)),
            pl.BlockSpec((FUSION, HID), lambda i: (0, 0)),
            pl.BlockSpec((1, HID), lambda i: (0, 0)),
            pl.BlockSpec((HID, NCLS), lambda i: (0, 0)),
            pl.BlockSpec((1, NCLS), lambda i: (0, 0)),
        ],
        out_specs=pl.BlockSpec((BM, NCLS), lambda i: (i, 0)),
        out_shape=jax.ShapeDtypeStruct((B, NCLS), jnp.float32),
    )(t1, t2, cat, num, wn, bn, w1, b1, w2, b2)


def kernel(text_title, text_query, cat0, cat1, cat2, cat3, cat4, cat5, cat6,
           cat7, numerical_inputs, emb_title, emb_query, emb_cat0, emb_cat1,
           emb_cat2, emb_cat3, emb_cat4, emb_cat5, emb_cat6, emb_cat7,
           W_num, b_num, W1, b1, W2, b2):
    tt = jnp.pad(text_title.astype(jnp.int32),
                 ((0, 0), (0, LPAD - L))).reshape(-1)
    tq = jnp.pad(text_query.astype(jnp.int32),
                 ((0, 0), (0, LPAD - L))).reshape(-1)
    cats = jnp.stack([cat0, cat1, cat2, cat3, cat4, cat5, cat6, cat7]
                     ).astype(jnp.int32).reshape(-1)
    t1 = _sc_text(tt, emb_title)
    t2 = _sc_text(tq, emb_query)
    cat = _sc_cats(cats, emb_cat0, emb_cat1, emb_cat2, emb_cat3,
                   emb_cat4, emb_cat5, emb_cat6, emb_cat7)
    return _mlp_call(t1, t2, cat, numerical_inputs,
                     W_num, b_num.reshape(1, NUMH),
                     W1, b1.reshape(1, HID),
                     W2, b2.reshape(1, NCLS))
